# baseline (speedup 1.0000x reference)
import jax
import jax.numpy as jnp
from jax.experimental import pallas as pl
from jax.experimental.pallas import tpu as pltpu


def _lstm_ae_kernel(x_ref, wih_e_ref, b_e_ref, whh_e_ref,
                    wih_d_ref, whh_d_ref, b_d_ref, out_ref):
    Bt, TI = x_ref.shape
    I = wih_e_ref.shape[0]
    H = whh_e_ref.shape[0]
    T = TI // I
    f32 = jnp.float32
    bf16 = jnp.bfloat16

    wih_e = wih_e_ref[...]
    whh_e = whh_e_ref[...]
    b_e = b_e_ref[...]

    h = jnp.zeros((Bt, H), f32)
    c = jnp.zeros((Bt, H), f32)
    for t in range(T):
        x_t = x_ref[:, t * I:(t + 1) * I].astype(bf16)
        gates = (jnp.dot(x_t, wih_e, preferred_element_type=f32) + b_e
                 + jnp.dot(h.astype(bf16), whh_e, preferred_element_type=f32))
        sig = jnp.tanh(gates[:, :3 * H]) * 0.5 + 0.5
        g_g = jnp.tanh(gates[:, 3 * H:])
        i_g = sig[:, 0 * H:1 * H]
        f_g = sig[:, 1 * H:2 * H]
        o_g = sig[:, 2 * H:3 * H]
        c = f_g * c + i_g * g_g
        h = o_g * jnp.tanh(c)

    xw_d = jnp.dot(h.astype(bf16), wih_d_ref[...],
                   preferred_element_type=f32) + b_d_ref[...]
    whh_d = whh_d_ref[...]

    hd = jnp.zeros((Bt, I), f32)
    cd = jnp.zeros((Bt, I), f32)
    for t in range(T):
        gates = xw_d + jnp.dot(hd.astype(bf16), whh_d,
                               preferred_element_type=f32)
        sig = jnp.tanh(gates[:, :3 * I]) * 0.5 + 0.5
        g_g = jnp.tanh(gates[:, 3 * I:])
        i_g = sig[:, 0 * I:1 * I]
        f_g = sig[:, 1 * I:2 * I]
        o_g = sig[:, 2 * I:3 * I]
        cd = f_g * cd + i_g * g_g
        hd = o_g * jnp.tanh(cd)
        out_ref[:, t * I:(t + 1) * I] = hd


def _prep_w(w, n):
    cols = jax.lax.broadcasted_iota(jnp.int32, w.shape, w.ndim - 1)
    return jnp.where(cols < 3 * n, w * 0.5, w).astype(jnp.bfloat16)


def _prep_b(b, n):
    cols = jax.lax.broadcasted_iota(jnp.int32, b.shape, b.ndim - 1)
    return jnp.where(cols < 3 * n, b * 0.5, b)


@jax.jit
def _forward(x, enc_wih_t, enc_b, enc_whh_t, dec_wih_t, dec_whh_t, dec_b):
    B, T, I = x.shape
    H = enc_whh_t.shape[0]
    f32 = jnp.float32

    x2 = x.reshape(B, T * I)
    wih_e = _prep_w(enc_wih_t, H)
    whh_e = _prep_w(enc_whh_t, H)
    b_e = _prep_b(enc_b, H)
    wih_d = _prep_w(dec_wih_t, I)
    whh_d = _prep_w(dec_whh_t, I)
    b_d = _prep_b(dec_b, I)

    bt = B // 2 if (B % 16 == 0) else B
    grid = (B // bt,)

    out_flat = pl.pallas_call(
        _lstm_ae_kernel,
        out_shape=jax.ShapeDtypeStruct((B, T * I), f32),
        grid=grid,
        in_specs=[
            pl.BlockSpec((bt, T * I), lambda b: (b, 0)),
            pl.BlockSpec((I, 4 * H), lambda b: (0, 0)),
            pl.BlockSpec((1, 4 * H), lambda b: (0, 0)),
            pl.BlockSpec((H, 4 * H), lambda b: (0, 0)),
            pl.BlockSpec((H, 4 * I), lambda b: (0, 0)),
            pl.BlockSpec((I, 4 * I), lambda b: (0, 0)),
            pl.BlockSpec((1, 4 * I), lambda b: (0, 0)),
        ],
        out_specs=pl.BlockSpec((bt, T * I), lambda b: (b, 0)),
        compiler_params=pltpu.CompilerParams(
            dimension_semantics=("parallel",),
            vmem_limit_bytes=64 * 1024 * 1024),
    )(x2, wih_e, b_e, whh_e, wih_d, whh_d, b_d)

    return out_flat.reshape(B, T, I)


def kernel(x, enc_wih_t, enc_b, enc_whh_t, dec_wih_t, dec_whh_t, dec_b):
    return _forward(x, enc_wih_t, enc_b, enc_whh_t, dec_wih_t,
                    dec_whh_t, dec_b)


# speedup vs baseline: 1.2197x; 1.2197x over previous
import jax
import jax.numpy as jnp
from jax.experimental import pallas as pl
from jax.experimental.pallas import tpu as pltpu


def _prep_kernel(x_ref, wih_e_ref, b_e_ref, whh_e_ref,
                 wih_d_ref, whh_d_ref, b_d_ref,
                 xt_ref, wih_e_o, b_e_o, whh_e_o, wih_d_o, whh_d_o, b_d_o):
    H = wih_e_ref.shape[1] // 4
    I = wih_d_ref.shape[1] // 4
    bf16 = jnp.bfloat16

    def scale(ref, n):
        v = ref[...]
        cols = jax.lax.broadcasted_iota(jnp.int32, v.shape, v.ndim - 1)
        return jnp.where(cols < 3 * n, v * 0.5, v)

    wih_e_o[...] = scale(wih_e_ref, H).astype(bf16)
    whh_e_o[...] = scale(whh_e_ref, H).astype(bf16)
    wih_d_o[...] = scale(wih_d_ref, I).astype(bf16)
    whh_d_o[...] = scale(whh_d_ref, I).astype(bf16)
    b_e_o[...] = scale(b_e_ref, H)
    b_d_o[...] = scale(b_d_ref, I)
    xt_ref[...] = jnp.transpose(x_ref[...], (1, 0, 2)).astype(bf16)


def _lstm_ae_kernel(xt_ref, wih_e_ref, b_e_ref, whh_e_ref,
                    wih_d_ref, whh_d_ref, b_d_ref, out_ref):
    T, Bt, I = xt_ref.shape
    H = whh_e_ref.shape[0]
    f32 = jnp.float32
    bf16 = jnp.bfloat16

    xw = jnp.dot(xt_ref[...].reshape(T * Bt, I), wih_e_ref[...],
                 preferred_element_type=f32) + b_e_ref[...]
    xw = xw.reshape(T, Bt, 4 * H)

    whh_e = whh_e_ref[...]

    h = jnp.zeros((Bt, H), f32)
    c = jnp.zeros((Bt, H), f32)
    for t in range(T):
        gates = xw[t] + jnp.dot(h.astype(bf16), whh_e,
                                preferred_element_type=f32)
        sig = jnp.tanh(gates[:, :3 * H]) * 0.5 + 0.5
        g_g = jnp.tanh(gates[:, 3 * H:])
        i_g = sig[:, 0 * H:1 * H]
        f_g = sig[:, 1 * H:2 * H]
        o_g = sig[:, 2 * H:3 * H]
        c = f_g * c + i_g * g_g
        h = o_g * jnp.tanh(c)

    xw_d = jnp.dot(h.astype(bf16), wih_d_ref[...],
                   preferred_element_type=f32) + b_d_ref[...]
    whh_d = whh_d_ref[...]

    hd = jnp.zeros((Bt, I), f32)
    cd = jnp.zeros((Bt, I), f32)
    for t in range(T):
        gates = xw_d + jnp.dot(hd.astype(bf16), whh_d,
                               preferred_element_type=f32)
        sig = jnp.tanh(gates[:, :3 * I]) * 0.5 + 0.5
        g_g = jnp.tanh(gates[:, 3 * I:])
        i_g = sig[:, 0 * I:1 * I]
        f_g = sig[:, 1 * I:2 * I]
        o_g = sig[:, 2 * I:3 * I]
        cd = f_g * cd + i_g * g_g
        hd = o_g * jnp.tanh(cd)
        out_ref[:, t * I:(t + 1) * I] = hd


@jax.jit
def _forward(x, enc_wih_t, enc_b, enc_whh_t, dec_wih_t, dec_whh_t, dec_b):
    B, T, I = x.shape
    H = enc_whh_t.shape[0]
    f32 = jnp.float32
    bf16 = jnp.bfloat16

    bt = B // 2 if (B % 16 == 0) else B
    ncores = B // bt

    half = lambda r: r // ncores

    xt, wih_e, b_e, whh_e, wih_d, whh_d, b_d = pl.pallas_call(
        _prep_kernel,
        out_shape=(
            jax.ShapeDtypeStruct((T, B, I), bf16),
            jax.ShapeDtypeStruct((I, 4 * H), bf16),
            jax.ShapeDtypeStruct((1, 4 * H), f32),
            jax.ShapeDtypeStruct((H, 4 * H), bf16),
            jax.ShapeDtypeStruct((H, 4 * I), bf16),
            jax.ShapeDtypeStruct((I, 4 * I), bf16),
            jax.ShapeDtypeStruct((1, 4 * I), f32),
        ),
        grid=(ncores,),
        in_specs=[
            pl.BlockSpec((bt, T, I), lambda b: (b, 0, 0)),
            pl.BlockSpec((half(I), 4 * H), lambda b: (b, 0)),
            pl.BlockSpec((1, 4 * H), lambda b: (0, 0)),
            pl.BlockSpec((half(H), 4 * H), lambda b: (b, 0)),
            pl.BlockSpec((half(H), 4 * I), lambda b: (b, 0)),
            pl.BlockSpec((half(I), 4 * I), lambda b: (b, 0)),
            pl.BlockSpec((1, 4 * I), lambda b: (0, 0)),
        ],
        out_specs=(
            pl.BlockSpec((T, bt, I), lambda b: (0, b, 0)),
            pl.BlockSpec((half(I), 4 * H), lambda b: (b, 0)),
            pl.BlockSpec((1, 4 * H), lambda b: (0, 0)),
            pl.BlockSpec((half(H), 4 * H), lambda b: (b, 0)),
            pl.BlockSpec((half(H), 4 * I), lambda b: (b, 0)),
            pl.BlockSpec((half(I), 4 * I), lambda b: (b, 0)),
            pl.BlockSpec((1, 4 * I), lambda b: (0, 0)),
        ),
        compiler_params=pltpu.CompilerParams(
            dimension_semantics=("parallel",),
            vmem_limit_bytes=64 * 1024 * 1024),
    )(x, enc_wih_t, enc_b, enc_whh_t, dec_wih_t, dec_whh_t, dec_b)

    out_flat = pl.pallas_call(
        _lstm_ae_kernel,
        out_shape=jax.ShapeDtypeStruct((B, T * I), f32),
        grid=(ncores,),
        in_specs=[
            pl.BlockSpec((T, bt, I), lambda b: (0, b, 0)),
            pl.BlockSpec((I, 4 * H), lambda b: (0, 0)),
            pl.BlockSpec((1, 4 * H), lambda b: (0, 0)),
            pl.BlockSpec((H, 4 * H), lambda b: (0, 0)),
            pl.BlockSpec((H, 4 * I), lambda b: (0, 0)),
            pl.BlockSpec((I, 4 * I), lambda b: (0, 0)),
            pl.BlockSpec((1, 4 * I), lambda b: (0, 0)),
        ],
        out_specs=pl.BlockSpec((bt, T * I), lambda b: (b, 0)),
        compiler_params=pltpu.CompilerParams(
            dimension_semantics=("parallel",),
            vmem_limit_bytes=64 * 1024 * 1024),
    )(xt, wih_e, b_e, whh_e, wih_d, whh_d, b_d)

    return out_flat.reshape(B, T, I)


def kernel(x, enc_wih_t, enc_b, enc_whh_t, dec_wih_t, dec_whh_t, dec_b):
    return _forward(x, enc_wih_t, enc_b, enc_whh_t, dec_wih_t,
                    dec_whh_t, dec_b)


# speedup vs baseline: 1.2319x; 1.0099x over previous
import jax
import jax.numpy as jnp
from jax.experimental import pallas as pl
from jax.experimental.pallas import tpu as pltpu


def _prep_kernel(x_ref, wih_e_ref, b_e_ref, whh_e_ref,
                 wih_d_ref, whh_d_ref, b_d_ref,
                 xt_ref, wih_e_o, b_e_o, whh_e_o, wih_d_o, whh_d_o, b_d_o):
    H = wih_e_ref.shape[1] // 4
    I = wih_d_ref.shape[1] // 4
    bf16 = jnp.bfloat16

    def scale(ref, n):
        v = ref[...]
        cols = jax.lax.broadcasted_iota(jnp.int32, v.shape, v.ndim - 1)
        return jnp.where(cols < 3 * n, v * 0.5, v)

    wih_e_o[...] = scale(wih_e_ref, H).astype(bf16)
    whh_e_o[...] = scale(whh_e_ref, H).astype(bf16)
    wih_d_o[...] = scale(wih_d_ref, I).astype(bf16)
    whh_d_o[...] = scale(whh_d_ref, I).astype(bf16)
    b_e_o[...] = scale(b_e_ref, H)
    b_d_o[...] = scale(b_d_ref, I)
    xt_ref[...] = jnp.transpose(x_ref[...], (1, 0, 2)).astype(bf16)


def _lstm_ae_kernel(xt_ref, wih_e_ref, b_e_ref, whh_e_ref,
                    wih_d_hbm, whh_d_hbm, b_d_ref, out_ref,
                    wih_d_v, whh_d_v, sems):
    T, Bt, I = xt_ref.shape
    H = whh_e_ref.shape[0]
    f32 = jnp.float32
    bf16 = jnp.bfloat16

    cp_wih_d = pltpu.make_async_copy(wih_d_hbm, wih_d_v, sems.at[0])
    cp_whh_d = pltpu.make_async_copy(whh_d_hbm, whh_d_v, sems.at[1])
    cp_wih_d.start()
    cp_whh_d.start()

    xw = jnp.dot(xt_ref[...].reshape(T * Bt, I), wih_e_ref[...],
                 preferred_element_type=f32) + b_e_ref[...]
    xw = xw.reshape(T, Bt, 4 * H)

    whh_e = whh_e_ref[...]

    h = jnp.zeros((Bt, H), f32)
    c = jnp.zeros((Bt, H), f32)
    for t in range(T):
        gates = xw[t] + jnp.dot(h.astype(bf16), whh_e,
                                preferred_element_type=f32)
        sig = jnp.tanh(gates[:, :3 * H]) * 0.5 + 0.5
        g_g = jnp.tanh(gates[:, 3 * H:])
        i_g = sig[:, 0 * H:1 * H]
        f_g = sig[:, 1 * H:2 * H]
        o_g = sig[:, 2 * H:3 * H]
        c = f_g * c + i_g * g_g
        h = o_g * jnp.tanh(c)

    cp_wih_d.wait()
    xw_d = jnp.dot(h.astype(bf16), wih_d_v[...],
                   preferred_element_type=f32) + b_d_ref[...]
    cp_whh_d.wait()
    whh_d = whh_d_v[...]

    hd = jnp.zeros((Bt, I), f32)
    cd = jnp.zeros((Bt, I), f32)
    for t in range(T):
        gates = xw_d + jnp.dot(hd.astype(bf16), whh_d,
                               preferred_element_type=f32)
        sig = jnp.tanh(gates[:, :3 * I]) * 0.5 + 0.5
        g_g = jnp.tanh(gates[:, 3 * I:])
        i_g = sig[:, 0 * I:1 * I]
        f_g = sig[:, 1 * I:2 * I]
        o_g = sig[:, 2 * I:3 * I]
        cd = f_g * cd + i_g * g_g
        hd = o_g * jnp.tanh(cd)
        out_ref[:, t * I:(t + 1) * I] = hd


@jax.jit
def _forward(x, enc_wih_t, enc_b, enc_whh_t, dec_wih_t, dec_whh_t, dec_b):
    B, T, I = x.shape
    H = enc_whh_t.shape[0]
    f32 = jnp.float32
    bf16 = jnp.bfloat16

    bt = B // 2 if (B % 16 == 0) else B
    ncores = B // bt

    half = lambda r: r // ncores

    xt, wih_e, b_e, whh_e, wih_d, whh_d, b_d = pl.pallas_call(
        _prep_kernel,
        out_shape=(
            jax.ShapeDtypeStruct((T, B, I), bf16),
            jax.ShapeDtypeStruct((I, 4 * H), bf16),
            jax.ShapeDtypeStruct((1, 4 * H), f32),
            jax.ShapeDtypeStruct((H, 4 * H), bf16),
            jax.ShapeDtypeStruct((H, 4 * I), bf16),
            jax.ShapeDtypeStruct((I, 4 * I), bf16),
            jax.ShapeDtypeStruct((1, 4 * I), f32),
        ),
        grid=(ncores,),
        in_specs=[
            pl.BlockSpec((bt, T, I), lambda b: (b, 0, 0)),
            pl.BlockSpec((half(I), 4 * H), lambda b: (b, 0)),
            pl.BlockSpec((1, 4 * H), lambda b: (0, 0)),
            pl.BlockSpec((half(H), 4 * H), lambda b: (b, 0)),
            pl.BlockSpec((half(H), 4 * I), lambda b: (b, 0)),
            pl.BlockSpec((half(I), 4 * I), lambda b: (b, 0)),
            pl.BlockSpec((1, 4 * I), lambda b: (0, 0)),
        ],
        out_specs=(
            pl.BlockSpec((T, bt, I), lambda b: (0, b, 0)),
            pl.BlockSpec((half(I), 4 * H), lambda b: (b, 0)),
            pl.BlockSpec((1, 4 * H), lambda b: (0, 0)),
            pl.BlockSpec((half(H), 4 * H), lambda b: (b, 0)),
            pl.BlockSpec((half(H), 4 * I), lambda b: (b, 0)),
            pl.BlockSpec((half(I), 4 * I), lambda b: (b, 0)),
            pl.BlockSpec((1, 4 * I), lambda b: (0, 0)),
        ),
        compiler_params=pltpu.CompilerParams(
            dimension_semantics=("parallel",),
            vmem_limit_bytes=64 * 1024 * 1024),
    )(x, enc_wih_t, enc_b, enc_whh_t, dec_wih_t, dec_whh_t, dec_b)

    out_flat = pl.pallas_call(
        _lstm_ae_kernel,
        out_shape=jax.ShapeDtypeStruct((B, T * I), f32),
        grid=(ncores,),
        in_specs=[
            pl.BlockSpec((T, bt, I), lambda b: (0, b, 0)),
            pl.BlockSpec((I, 4 * H), lambda b: (0, 0)),
            pl.BlockSpec((1, 4 * H), lambda b: (0, 0)),
            pl.BlockSpec((H, 4 * H), lambda b: (0, 0)),
            pl.BlockSpec(memory_space=pl.ANY),
            pl.BlockSpec(memory_space=pl.ANY),
            pl.BlockSpec((1, 4 * I), lambda b: (0, 0)),
        ],
        out_specs=pl.BlockSpec((bt, T * I), lambda b: (b, 0)),
        scratch_shapes=[
            pltpu.VMEM((H, 4 * I), bf16),
            pltpu.VMEM((I, 4 * I), bf16),
            pltpu.SemaphoreType.DMA((2,)),
        ],
        compiler_params=pltpu.CompilerParams(
            dimension_semantics=("parallel",),
            vmem_limit_bytes=64 * 1024 * 1024),
    )(xt, wih_e, b_e, whh_e, wih_d, whh_d, b_d)

    return out_flat.reshape(B, T, I)


def kernel(x, enc_wih_t, enc_b, enc_whh_t, dec_wih_t, dec_whh_t, dec_b):
    return _forward(x, enc_wih_t, enc_b, enc_whh_t, dec_wih_t,
                    dec_whh_t, dec_b)


# speedup vs baseline: 1.3031x; 1.0578x over previous
import jax
import jax.numpy as jnp
from jax.experimental import pallas as pl
from jax.experimental.pallas import tpu as pltpu


def _prep_kernel(x_ref, wih_e_ref, b_e_ref, whh_e_ref,
                 wih_d_ref, whh_d_ref, b_d_ref,
                 xt_ref, wih_e_o, b_e_o, whh_e_o, wih_d_o, whh_d_o, b_d_o):
    H = wih_e_ref.shape[1] // 4
    I = wih_d_ref.shape[1] // 4
    bf16 = jnp.bfloat16

    def scale(ref, n):
        v = ref[...]
        cols = jax.lax.broadcasted_iota(jnp.int32, v.shape, v.ndim - 1)
        return jnp.where(cols < 3 * n, v * 0.5, v)

    wih_e_o[...] = scale(wih_e_ref, H).astype(bf16)
    whh_e_o[...] = scale(whh_e_ref, H).astype(bf16)
    wih_d_o[...] = scale(wih_d_ref, I).astype(bf16)
    whh_d_o[...] = scale(whh_d_ref, I).astype(bf16)
    b_e_o[...] = scale(b_e_ref, H)
    b_d_o[...] = scale(b_d_ref, I)
    xt_ref[...] = jnp.transpose(x_ref[...], (1, 0, 2)).astype(bf16)


def _lstm_ae_kernel(xt_ref, wih_e_ref, b_e_ref, whh_e_ref,
                    wih_d_ref, whh_d_ref, b_d_ref, out_ref):
    T, Bt, I = xt_ref.shape
    H = whh_e_ref.shape[0]
    f32 = jnp.float32
    bf16 = jnp.bfloat16

    xw = jnp.dot(xt_ref[...].reshape(T * Bt, I), wih_e_ref[...],
                 preferred_element_type=f32) + b_e_ref[...]
    xw = xw.reshape(T, Bt, 4 * H)

    whh_e = whh_e_ref[...]

    h = jnp.zeros((Bt, H), f32)
    c = jnp.zeros((Bt, H), f32)
    for t in range(T):
        gates = xw[t] + jnp.dot(h.astype(bf16), whh_e,
                                preferred_element_type=f32)
        sig = jnp.tanh(gates[:, :3 * H]) * 0.5 + 0.5
        g_g = jnp.tanh(gates[:, 3 * H:])
        i_g = sig[:, 0 * H:1 * H]
        f_g = sig[:, 1 * H:2 * H]
        o_g = sig[:, 2 * H:3 * H]
        c = f_g * c + i_g * g_g
        h = o_g * jnp.tanh(c)

    xw_d = jnp.dot(h.astype(bf16), wih_d_ref[...],
                   preferred_element_type=f32) + b_d_ref[...]
    whh_d = whh_d_ref[...]

    hd = jnp.zeros((Bt, I), f32)
    cd = jnp.zeros((Bt, I), f32)
    for t in range(T):
        gates = xw_d + jnp.dot(hd.astype(bf16), whh_d,
                               preferred_element_type=f32)
        sig = jnp.tanh(gates[:, :3 * I]) * 0.5 + 0.5
        g_g = jnp.tanh(gates[:, 3 * I:])
        i_g = sig[:, 0 * I:1 * I]
        f_g = sig[:, 1 * I:2 * I]
        o_g = sig[:, 2 * I:3 * I]
        cd = f_g * cd + i_g * g_g
        hd = o_g * jnp.tanh(cd)
        out_ref[:, t * I:(t + 1) * I] = hd


@jax.jit
def _forward(x, enc_wih_t, enc_b, enc_whh_t, dec_wih_t, dec_whh_t, dec_b):
    B, T, I = x.shape
    H = enc_whh_t.shape[0]
    f32 = jnp.float32
    bf16 = jnp.bfloat16

    bt = B // 2 if (B % 16 == 0) else B
    ncores = B // bt

    half = lambda r: r // ncores

    xt, wih_e, b_e, whh_e, wih_d, whh_d, b_d = pl.pallas_call(
        _prep_kernel,
        out_shape=(
            jax.ShapeDtypeStruct((T, B, I), bf16),
            jax.ShapeDtypeStruct((I, 4 * H), bf16),
            jax.ShapeDtypeStruct((1, 4 * H), f32),
            jax.ShapeDtypeStruct((H, 4 * H), bf16),
            jax.ShapeDtypeStruct((H, 4 * I), bf16),
            jax.ShapeDtypeStruct((I, 4 * I), bf16),
            jax.ShapeDtypeStruct((1, 4 * I), f32),
        ),
        grid=(ncores,),
        in_specs=[
            pl.BlockSpec((bt, T, I), lambda b: (b, 0, 0)),
            pl.BlockSpec((half(I), 4 * H), lambda b: (b, 0)),
            pl.BlockSpec((1, 4 * H), lambda b: (0, 0)),
            pl.BlockSpec((half(H), 4 * H), lambda b: (b, 0)),
            pl.BlockSpec((half(H), 4 * I), lambda b: (b, 0)),
            pl.BlockSpec((half(I), 4 * I), lambda b: (b, 0)),
            pl.BlockSpec((1, 4 * I), lambda b: (0, 0)),
        ],
        out_specs=(
            pl.BlockSpec((T, bt, I), lambda b: (0, b, 0)),
            pl.BlockSpec((half(I), 4 * H), lambda b: (b, 0)),
            pl.BlockSpec((1, 4 * H), lambda b: (0, 0)),
            pl.BlockSpec((half(H), 4 * H), lambda b: (b, 0)),
            pl.BlockSpec((half(H), 4 * I), lambda b: (b, 0)),
            pl.BlockSpec((half(I), 4 * I), lambda b: (b, 0)),
            pl.BlockSpec((1, 4 * I), lambda b: (0, 0)),
        ),
        compiler_params=pltpu.CompilerParams(
            dimension_semantics=("parallel",),
            vmem_limit_bytes=16 * 1024 * 1024),
    )(x, enc_wih_t, enc_b, enc_whh_t, dec_wih_t, dec_whh_t, dec_b)

    out_flat = pl.pallas_call(
        _lstm_ae_kernel,
        out_shape=jax.ShapeDtypeStruct((B, T * I), f32),
        grid=(ncores,),
        in_specs=[
            pl.BlockSpec((T, bt, I), lambda b: (0, b, 0)),
            pl.BlockSpec((I, 4 * H), lambda b: (0, 0)),
            pl.BlockSpec((1, 4 * H), lambda b: (0, 0)),
            pl.BlockSpec((H, 4 * H), lambda b: (0, 0)),
            pl.BlockSpec((H, 4 * I), lambda b: (0, 0)),
            pl.BlockSpec((I, 4 * I), lambda b: (0, 0)),
            pl.BlockSpec((1, 4 * I), lambda b: (0, 0)),
        ],
        out_specs=pl.BlockSpec((bt, T * I), lambda b: (b, 0)),
        compiler_params=pltpu.CompilerParams(
            dimension_semantics=("parallel",),
            vmem_limit_bytes=26 * 1024 * 1024),
    )(xt, wih_e, b_e, whh_e, wih_d, whh_d, b_d)

    return out_flat.reshape(B, T, I)


def kernel(x, enc_wih_t, enc_b, enc_whh_t, dec_wih_t, dec_whh_t, dec_b):
    return _forward(x, enc_wih_t, enc_b, enc_whh_t, dec_wih_t,
                    dec_whh_t, dec_b)
